# baseline (speedup 1.0000x reference)
import functools

import jax
import jax.numpy as jnp
from jax.experimental import pallas as pl

_B, _NPB = 4, 4096
_STRIDE, _NSAMPLE = 4, 16
_IN, _OUT = 256, 512
_MPB = _NPB // _STRIDE + 1
_N = _B * _NPB
_M = _B * _MPB


def _matmul_kernel(a_ref, w_ref, o_ref):
    o_ref[...] = jnp.dot(a_ref[...], w_ref[...],
                         preferred_element_type=jnp.float32)


def _dense_u(p, x, W):
    a = jnp.concatenate([p, x], axis=1)
    a = jnp.pad(a, ((0, 0), (0, 384 - (3 + _IN))))
    w = jnp.pad(W, ((0, 384 - (3 + _IN)), (0, 0)))
    return pl.pallas_call(
        _matmul_kernel,
        grid=(_N // 1024,),
        in_specs=[pl.BlockSpec((1024, 384), lambda i: (i, 0)),
                  pl.BlockSpec((384, _OUT), lambda i: (0, 0))],
        out_specs=pl.BlockSpec((1024, _OUT), lambda i: (i, 0)),
        out_shape=jax.ShapeDtypeStruct((_N, _OUT), jnp.float32),
    )(a, w)


def _fps_kernel(px_ref, py_ref, pz_ref, idx_ref):
    px, py, pz = px_ref[...], py_ref[...], pz_ref[...]
    lane = jax.lax.broadcasted_iota(jnp.int32, (_B, _NPB), 1)
    out_lane = jax.lax.broadcasted_iota(jnp.int32, (_B, _MPB), 1)

    qx0, qy0, qz0 = px[:, 0:1], py[:, 0:1], pz[:, 0:1]
    d0 = (px - qx0) ** 2 + (py - qy0) ** 2 + (pz - qz0) ** 2
    idx_ref[...] = jnp.zeros((_B, _MPB), jnp.int32)

    def step(i, dists):
        m = jnp.max(dists, axis=1, keepdims=True)
        cand = jnp.where(dists == m, lane, _NPB)
        nxt = jnp.min(cand, axis=1, keepdims=True)
        sel = lane == nxt
        qx = jnp.sum(jnp.where(sel, px, 0.0), axis=1, keepdims=True)
        qy = jnp.sum(jnp.where(sel, py, 0.0), axis=1, keepdims=True)
        qz = jnp.sum(jnp.where(sel, pz, 0.0), axis=1, keepdims=True)
        d = (px - qx) ** 2 + (py - qy) ** 2 + (pz - qz) ** 2
        idx_ref[...] = jnp.where(out_lane == i, nxt, idx_ref[...])
        return jnp.minimum(dists, d)

    jax.lax.fori_loop(1, _MPB, step, d0, unroll=False)


def _fps_all(pb):
    px = pb[:, :, 0]
    py = pb[:, :, 1]
    pz = pb[:, :, 2]
    return pl.pallas_call(
        _fps_kernel,
        out_shape=jax.ShapeDtypeStruct((_B, _MPB), jnp.int32),
    )(px, py, pz)


_QR = 128
_MPAD = 1152


def _knn_kernel(qx_ref, qy_ref, qz_ref, px_ref, py_ref, pz_ref, nidx_ref):
    qx = qx_ref[0]
    qy = qy_ref[0]
    qz = qz_ref[0]
    px = px_ref[0]
    py = py_ref[0]
    pz = pz_ref[0]
    d = (qx - px) ** 2 + (qy - py) ** 2 + (qz - pz) ** 2
    lane = jax.lax.broadcasted_iota(jnp.int32, (_QR, _NPB), 1)
    big = jnp.int32(2 ** 30)
    for k in range(_NSAMPLE):
        m = jnp.min(d, axis=1, keepdims=True)
        nxt = jnp.min(jnp.where(d == m, lane, big), axis=1, keepdims=True)
        nidx_ref[0, :, k:k + 1] = nxt
        d = jnp.where(lane == nxt, jnp.inf, d)


def _knn_all(new_xyz_pad, pb):
    q = [new_xyz_pad[:, :, c:c + 1] for c in range(3)]
    pt = [pb[:, :, c].reshape(_B, 1, _NPB) for c in range(3)]
    qspec = pl.BlockSpec((1, _QR, 1), lambda b, i: (b, i, 0))
    pspec = pl.BlockSpec((1, 1, _NPB), lambda b, i: (b, 0, 0))
    return pl.pallas_call(
        _knn_kernel,
        grid=(_B, _MPAD // _QR),
        in_specs=[qspec, qspec, qspec, pspec, pspec, pspec],
        out_specs=pl.BlockSpec((1, _QR, _NSAMPLE), lambda b, i: (b, i, 0)),
        out_shape=jax.ShapeDtypeStruct((_B, _MPAD, _NSAMPLE), jnp.int32),
    )(*q, *pt)


def kernel(p, x, o, condition, W, cond_gamma, cond_beta):
    pb = p.reshape(_B, _NPB, 3)
    idx = _fps_all(pb)
    new_xyz = jnp.take_along_axis(pb, idx[:, :, None], axis=1)

    new_xyz_pad = jnp.pad(new_xyz, ((0, 0), (0, _MPAD - _MPB), (0, 0)))
    nidx = _knn_all(new_xyz_pad, pb)[:, :_MPB]
    gidx = (nidx + (jnp.arange(_B, dtype=jnp.int32) * _NPB)[:, None, None])
    gidx = gidx.reshape(_M, _NSAMPLE)

    u = _dense_u(p, x, W)

    c = new_xyz.reshape(_M, 3) @ W[:3]

    ug = u[gidx]
    h = ug - c[:, None, :]
    hmax = jnp.max(h, axis=1)
    hmin = jnp.min(h, axis=1)
    s1 = jnp.sum(h, axis=(0, 1))
    s2 = jnp.sum(h * h, axis=(0, 1))

    cnt = _M * _NSAMPLE
    mean = s1 / cnt
    var = s2 / cnt - mean * mean
    gamma = cond_gamma[condition]
    beta = cond_beta[condition]
    scale = gamma / jnp.sqrt(var + 1e-5)
    bias = beta - mean * scale
    hsel = jnp.where(scale >= 0, hmax, hmin)
    out = jax.nn.relu(hsel * scale[None, :] + bias[None, :])

    n_p = new_xyz.reshape(_M, 3)
    n_o = jnp.array([(i + 1) * _MPB for i in range(_B)], jnp.int32)
    return (n_p, out, n_o)


# speedup vs baseline: 1.0704x; 1.0704x over previous
import functools

import jax
import jax.numpy as jnp
from jax import lax
from jax.experimental import pallas as pl
from jax.experimental.pallas import tpu as pltpu
from jax.experimental.pallas import tpu_sc as plsc

_B, _NPB = 4, 4096
_STRIDE, _NSAMPLE = 4, 16
_IN, _OUT = 256, 512
_MPB = _NPB // _STRIDE + 1
_N = _B * _NPB
_M = _B * _MPB


def _matmul_kernel(a_ref, w_ref, o_ref):
    o_ref[...] = jnp.dot(a_ref[...], w_ref[...],
                         preferred_element_type=jnp.float32)


def _dense_u(p, x, W):
    a = jnp.concatenate([p, x], axis=1)
    a = jnp.pad(a, ((0, 0), (0, 384 - (3 + _IN))))
    w = jnp.pad(W, ((0, 384 - (3 + _IN)), (0, 0)))
    return pl.pallas_call(
        _matmul_kernel,
        grid=(_N // 1024,),
        in_specs=[pl.BlockSpec((1024, 384), lambda i: (i, 0)),
                  pl.BlockSpec((384, _OUT), lambda i: (0, 0))],
        out_specs=pl.BlockSpec((1024, _OUT), lambda i: (i, 0)),
        out_shape=jax.ShapeDtypeStruct((_N, _OUT), jnp.float32),
    )(a, w)


def _fps_kernel(px_ref, py_ref, pz_ref, idx_ref):
    px, py, pz = px_ref[...], py_ref[...], pz_ref[...]
    lane = jax.lax.broadcasted_iota(jnp.int32, (_B, _NPB), 1)
    out_lane = jax.lax.broadcasted_iota(jnp.int32, (_B, _MPB), 1)

    qx0, qy0, qz0 = px[:, 0:1], py[:, 0:1], pz[:, 0:1]
    d0 = ((px - qx0) ** 2 + (pz - qz0) ** 2) + (py - qy0) ** 2
    idx_ref[...] = jnp.zeros((_B, _MPB), jnp.int32)

    def step(i, dists):
        m = jnp.max(dists, axis=1, keepdims=True)
        cand = jnp.where(dists == m, lane, _NPB)
        nxt = jnp.min(cand, axis=1, keepdims=True)
        sel = lane == nxt
        qx = jnp.sum(jnp.where(sel, px, 0.0), axis=1, keepdims=True)
        qy = jnp.sum(jnp.where(sel, py, 0.0), axis=1, keepdims=True)
        qz = jnp.sum(jnp.where(sel, pz, 0.0), axis=1, keepdims=True)
        d = ((px - qx) ** 2 + (pz - qz) ** 2) + (py - qy) ** 2
        idx_ref[...] = jnp.where(out_lane == i, nxt, idx_ref[...])
        return jnp.minimum(dists, d)

    jax.lax.fori_loop(1, _MPB, step, d0, unroll=False)


def _fps_all(pb):
    px = pb[:, :, 0]
    py = pb[:, :, 1]
    pz = pb[:, :, 2]
    return pl.pallas_call(
        _fps_kernel,
        out_shape=jax.ShapeDtypeStruct((_B, _MPB), jnp.int32),
    )(px, py, pz)


_QR = 128
_MPAD = 1152


def _knn_kernel(qx_ref, qy_ref, qz_ref, px_ref, py_ref, pz_ref, nidx_ref):
    qx = qx_ref[0]
    qy = qy_ref[0]
    qz = qz_ref[0]
    px = px_ref[0]
    py = py_ref[0]
    pz = pz_ref[0]
    d = (qx - px) ** 2 + (qy - py) ** 2 + (qz - pz) ** 2
    lane = jax.lax.broadcasted_iota(jnp.int32, (_QR, _NPB), 1)
    big = jnp.int32(2 ** 30)
    for k in range(_NSAMPLE):
        m = jnp.min(d, axis=1, keepdims=True)
        nxt = jnp.min(jnp.where(d == m, lane, big), axis=1, keepdims=True)
        nidx_ref[0, :, k:k + 1] = nxt
        d = jnp.where(lane == nxt, jnp.inf, d)


def _knn_all(new_xyz_pad, pb):
    q = [new_xyz_pad[:, :, c:c + 1] for c in range(3)]
    pt = [pb[:, :, c].reshape(_B, 1, _NPB) for c in range(3)]
    qspec = pl.BlockSpec((1, _QR, 1), lambda b, i: (b, i, 0))
    pspec = pl.BlockSpec((1, 1, _NPB), lambda b, i: (b, 0, 0))
    return pl.pallas_call(
        _knn_kernel,
        grid=(_B, _MPAD // _QR),
        in_specs=[qspec, qspec, qspec, pspec, pspec, pspec],
        out_specs=pl.BlockSpec((1, _QR, _NSAMPLE), lambda b, i: (b, i, 0)),
        out_shape=jax.ShapeDtypeStruct((_B, _MPAD, _NSAMPLE), jnp.int32),
    )(*q, *pt)


_NW = 32
_RPW = 136
_MP = _NW * _RPW
_GRP = 8


def _sc_body(u_hbm, gidx_hbm, mx_hbm, mn_hbm, s1_hbm, s2_hbm,
             idx_v, rows_v, st_mx, st_mn, st_s1, st_s2, sem0, sem1):
    wid = lax.axis_index("s") * 2 + lax.axis_index("c")
    base = wid * _RPW
    pltpu.sync_copy(gidx_hbm.at[pl.ds(base, _RPW)], idx_v)
    pltpu.async_copy(u_hbm.at[idx_v.at[0]], rows_v.at[0], sem0)

    def compute(m, slot, sem):
        pltpu.make_async_copy(u_hbm.at[idx_v.at[m]], rows_v.at[slot], sem).wait()
        sm = m % _GRP

        def chunk(ci, _):
            off = ci * 16
            v = rows_v[slot, 0, pl.ds(off, 16)]
            mx = v
            mn = v
            s1 = v
            s2 = v * v
            for r in range(1, _NSAMPLE):
                v = rows_v[slot, r, pl.ds(off, 16)]
                mx = jnp.maximum(mx, v)
                mn = jnp.minimum(mn, v)
                s1 = s1 + v
                s2 = s2 + v * v
            st_mx[sm, pl.ds(off, 16)] = mx
            st_mn[sm, pl.ds(off, 16)] = mn
            st_s1[sm, pl.ds(off, 16)] = s1
            st_s2[sm, pl.ds(off, 16)] = s2
            return 0

        lax.fori_loop(0, _OUT // 16, chunk, 0)

        @pl.when(sm == _GRP - 1)
        def _flush():
            row0 = pl.multiple_of(base + m - (_GRP - 1), _GRP)
            pltpu.sync_copy(st_mx, mx_hbm.at[pl.ds(row0, _GRP)])
            pltpu.sync_copy(st_mn, mn_hbm.at[pl.ds(row0, _GRP)])
            pltpu.sync_copy(st_s1, s1_hbm.at[pl.ds(row0, _GRP)])
            pltpu.sync_copy(st_s2, s2_hbm.at[pl.ds(row0, _GRP)])

    def pair(g2, _):
        m0 = g2 * 2

        @pl.when(m0 + 1 < _RPW)
        def _start1():
            pltpu.async_copy(u_hbm.at[idx_v.at[m0 + 1]], rows_v.at[1], sem1)

        compute(m0, 0, sem0)

        @pl.when(m0 + 2 < _RPW)
        def _start0():
            pltpu.async_copy(u_hbm.at[idx_v.at[m0 + 2]], rows_v.at[0], sem0)

        compute(m0 + 1, 1, sem1)
        return 0

    lax.fori_loop(0, _RPW // 2, pair, 0)


def _sc_gather_reduce(u, gidx_pad):
    f32 = jnp.float32
    out = jax.ShapeDtypeStruct((_MP, _OUT), f32)
    run = functools.partial(
        pl.kernel,
        mesh=plsc.VectorSubcoreMesh(core_axis_name="c", subcore_axis_name="s"),
        out_type=[out, out, out, out],
        scratch_types=[
            pltpu.VMEM((_RPW, _NSAMPLE), jnp.int32),
            pltpu.VMEM((2, _NSAMPLE, _OUT), f32),
            pltpu.VMEM((_GRP, _OUT), f32),
            pltpu.VMEM((_GRP, _OUT), f32),
            pltpu.VMEM((_GRP, _OUT), f32),
            pltpu.VMEM((_GRP, _OUT), f32),
            pltpu.SemaphoreType.DMA,
            pltpu.SemaphoreType.DMA,
        ],
    )(_sc_body)
    return run(u, gidx_pad)


def kernel(p, x, o, condition, W, cond_gamma, cond_beta):
    pb = p.reshape(_B, _NPB, 3)
    idx = _fps_all(pb)
    new_xyz = jnp.take_along_axis(pb, idx[:, :, None], axis=1)

    new_xyz_pad = jnp.pad(new_xyz, ((0, 0), (0, _MPAD - _MPB), (0, 0)))
    nidx = _knn_all(new_xyz_pad, pb)[:, :_MPB]
    gidx = (nidx + (jnp.arange(_B, dtype=jnp.int32) * _NPB)[:, None, None])
    gidx = gidx.reshape(_M, _NSAMPLE)

    u = _dense_u(p, x, W)

    c = new_xyz.reshape(_M, 3) @ W[:3]

    gidx_pad = jnp.pad(gidx, ((0, _MP - _M), (0, 0)))
    gmx, gmn, gs1, gs2 = _sc_gather_reduce(u, gidx_pad)
    gmx, gmn, gs1, gs2 = gmx[:_M], gmn[:_M], gs1[:_M], gs2[:_M]

    cnt = _M * _NSAMPLE
    s1 = jnp.sum(gs1 - _NSAMPLE * c, axis=0)
    s2 = jnp.sum(gs2 - 2.0 * c * gs1 + _NSAMPLE * c * c, axis=0)
    mean = s1 / cnt
    var = s2 / cnt - mean * mean
    gamma = cond_gamma[condition]
    beta = cond_beta[condition]
    scale = gamma / jnp.sqrt(var + 1e-5)
    bias = beta - mean * scale
    gsel = jnp.where(scale >= 0, gmx, gmn)
    out = jax.nn.relu((gsel - c) * scale[None, :] + bias[None, :])

    n_p = new_xyz.reshape(_M, 3)
    n_o = jnp.array([(i + 1) * _MPB for i in range(_B)], jnp.int32)
    return (n_p, out, n_o)


# speedup vs baseline: 1.0784x; 1.0074x over previous
import functools

import jax
import jax.numpy as jnp
from jax import lax
from jax.experimental import pallas as pl
from jax.experimental.pallas import tpu as pltpu
from jax.experimental.pallas import tpu_sc as plsc

_B, _NPB = 4, 4096
_STRIDE, _NSAMPLE = 4, 16
_IN, _OUT = 256, 512
_MPB = _NPB // _STRIDE + 1
_N = _B * _NPB
_M = _B * _MPB


def _matmul_kernel(x_ref, p_ref, wf_ref, w3_ref, o_ref):
    o_ref[...] = (jnp.dot(x_ref[...], wf_ref[...],
                          preferred_element_type=jnp.float32)
                  + jnp.dot(p_ref[...], w3_ref[...],
                            preferred_element_type=jnp.float32))


def _dense_u(p, x, W):
    p8 = jnp.pad(p, ((0, 0), (0, 5)))
    w3 = jnp.pad(W[:3], ((0, 5), (0, 0)))
    return pl.pallas_call(
        _matmul_kernel,
        grid=(_N // 1024,),
        in_specs=[pl.BlockSpec((1024, _IN), lambda i: (i, 0)),
                  pl.BlockSpec((1024, 8), lambda i: (i, 0)),
                  pl.BlockSpec((_IN, _OUT), lambda i: (0, 0)),
                  pl.BlockSpec((8, _OUT), lambda i: (0, 0))],
        out_specs=pl.BlockSpec((1024, _OUT), lambda i: (i, 0)),
        out_shape=jax.ShapeDtypeStruct((_N, _OUT), jnp.float32),
    )(x, p8, W[3:], w3)


def _fps_kernel(px_ref, py_ref, pz_ref, idx_ref):
    px, py, pz = px_ref[...], py_ref[...], pz_ref[...]
    lane = jax.lax.broadcasted_iota(jnp.int32, (_B, _NPB), 1)
    out_lane = jax.lax.broadcasted_iota(jnp.int32, (_B, _MPB), 1)

    qx0, qy0, qz0 = px[:, 0:1], py[:, 0:1], pz[:, 0:1]
    d0 = ((px - qx0) ** 2 + (pz - qz0) ** 2) + (py - qy0) ** 2
    idx_ref[...] = jnp.zeros((_B, _MPB), jnp.int32)

    def step(i, dists):
        m = jnp.max(dists, axis=1, keepdims=True)
        cand = jnp.where(dists == m, lane, _NPB)
        nxt = jnp.min(cand, axis=1, keepdims=True)
        sel = lane == nxt
        qx = jnp.sum(jnp.where(sel, px, 0.0), axis=1, keepdims=True)
        qy = jnp.sum(jnp.where(sel, py, 0.0), axis=1, keepdims=True)
        qz = jnp.sum(jnp.where(sel, pz, 0.0), axis=1, keepdims=True)
        d = ((px - qx) ** 2 + (pz - qz) ** 2) + (py - qy) ** 2
        idx_ref[...] = jnp.where(out_lane == i, nxt, idx_ref[...])
        return jnp.minimum(dists, d)

    jax.lax.fori_loop(1, _MPB, step, d0, unroll=False)


def _fps_all(pb):
    px = pb[:, :, 0]
    py = pb[:, :, 1]
    pz = pb[:, :, 2]
    return pl.pallas_call(
        _fps_kernel,
        out_shape=jax.ShapeDtypeStruct((_B, _MPB), jnp.int32),
    )(px, py, pz)


_QR = 128
_MPAD = 1152


def _knn_kernel(qx_ref, qy_ref, qz_ref, px_ref, py_ref, pz_ref, nidx_ref):
    qx = qx_ref[0]
    qy = qy_ref[0]
    qz = qz_ref[0]
    px = px_ref[0]
    py = py_ref[0]
    pz = pz_ref[0]
    d = (qx - px) ** 2 + (qy - py) ** 2 + (qz - pz) ** 2
    lane = jax.lax.broadcasted_iota(jnp.int32, (_QR, _NPB), 1)
    big = jnp.int32(2 ** 30)
    for k in range(_NSAMPLE):
        m = jnp.min(d, axis=1, keepdims=True)
        nxt = jnp.min(jnp.where(d == m, lane, big), axis=1, keepdims=True)
        nidx_ref[0, :, k:k + 1] = nxt
        d = jnp.where(lane == nxt, jnp.inf, d)


def _knn_all(new_xyz_pad, pb):
    q = [new_xyz_pad[:, :, c:c + 1] for c in range(3)]
    pt = [pb[:, :, c].reshape(_B, 1, _NPB) for c in range(3)]
    qspec = pl.BlockSpec((1, _QR, 1), lambda b, i: (b, i, 0))
    pspec = pl.BlockSpec((1, 1, _NPB), lambda b, i: (b, 0, 0))
    return pl.pallas_call(
        _knn_kernel,
        grid=(_B, _MPAD // _QR),
        in_specs=[qspec, qspec, qspec, pspec, pspec, pspec],
        out_specs=pl.BlockSpec((1, _QR, _NSAMPLE), lambda b, i: (b, i, 0)),
        out_shape=jax.ShapeDtypeStruct((_B, _MPAD, _NSAMPLE), jnp.int32),
    )(*q, *pt)


_NW = 32
_RPW = 136
_MP = _NW * _RPW
_GRP = 8


def _sc_body(u_hbm, gidx_hbm, mx_hbm, mn_hbm, s1_hbm, s2_hbm,
             idx_v, rows_v, st_mx, st_mn, st_s1, st_s2, sem0, sem1):
    wid = lax.axis_index("s") * 2 + lax.axis_index("c")
    base = wid * _RPW
    pltpu.sync_copy(gidx_hbm.at[pl.ds(base, _RPW)], idx_v)
    pltpu.async_copy(u_hbm.at[idx_v.at[0]], rows_v.at[0], sem0)

    def compute(m, slot, sem):
        pltpu.make_async_copy(u_hbm.at[idx_v.at[m]], rows_v.at[slot], sem).wait()
        sm = m % _GRP

        def chunk(ci, _):
            off = ci * 16
            v = rows_v[slot, 0, pl.ds(off, 16)]
            mx = v
            mn = v
            s1 = v
            s2 = v * v
            for r in range(1, _NSAMPLE):
                v = rows_v[slot, r, pl.ds(off, 16)]
                mx = jnp.maximum(mx, v)
                mn = jnp.minimum(mn, v)
                s1 = s1 + v
                s2 = s2 + v * v
            st_mx[sm, pl.ds(off, 16)] = mx
            st_mn[sm, pl.ds(off, 16)] = mn
            st_s1[sm, pl.ds(off, 16)] = s1
            st_s2[sm, pl.ds(off, 16)] = s2
            return 0

        lax.fori_loop(0, _OUT // 16, chunk, 0)

        @pl.when(sm == _GRP - 1)
        def _flush():
            row0 = pl.multiple_of(base + m - (_GRP - 1), _GRP)
            pltpu.sync_copy(st_mx, mx_hbm.at[pl.ds(row0, _GRP)])
            pltpu.sync_copy(st_mn, mn_hbm.at[pl.ds(row0, _GRP)])
            pltpu.sync_copy(st_s1, s1_hbm.at[pl.ds(row0, _GRP)])
            pltpu.sync_copy(st_s2, s2_hbm.at[pl.ds(row0, _GRP)])

    def pair(g2, _):
        m0 = g2 * 2

        @pl.when(m0 + 1 < _RPW)
        def _start1():
            pltpu.async_copy(u_hbm.at[idx_v.at[m0 + 1]], rows_v.at[1], sem1)

        compute(m0, 0, sem0)

        @pl.when(m0 + 2 < _RPW)
        def _start0():
            pltpu.async_copy(u_hbm.at[idx_v.at[m0 + 2]], rows_v.at[0], sem0)

        compute(m0 + 1, 1, sem1)
        return 0

    lax.fori_loop(0, _RPW // 2, pair, 0)


def _sc_gather_reduce(u, gidx_pad):
    f32 = jnp.float32
    out = jax.ShapeDtypeStruct((_MP, _OUT), f32)
    run = functools.partial(
        pl.kernel,
        mesh=plsc.VectorSubcoreMesh(core_axis_name="c", subcore_axis_name="s"),
        out_type=[out, out, out, out],
        scratch_types=[
            pltpu.VMEM((_RPW, _NSAMPLE), jnp.int32),
            pltpu.VMEM((2, _NSAMPLE, _OUT), f32),
            pltpu.VMEM((_GRP, _OUT), f32),
            pltpu.VMEM((_GRP, _OUT), f32),
            pltpu.VMEM((_GRP, _OUT), f32),
            pltpu.VMEM((_GRP, _OUT), f32),
            pltpu.SemaphoreType.DMA,
            pltpu.SemaphoreType.DMA,
        ],
    )(_sc_body)
    return run(u, gidx_pad)


def kernel(p, x, o, condition, W, cond_gamma, cond_beta):
    pb = p.reshape(_B, _NPB, 3)
    idx = _fps_all(pb)
    new_xyz = jnp.take_along_axis(pb, idx[:, :, None], axis=1)

    new_xyz_pad = jnp.pad(new_xyz, ((0, 0), (0, _MPAD - _MPB), (0, 0)))
    nidx = _knn_all(new_xyz_pad, pb)[:, :_MPB]
    gidx = (nidx + (jnp.arange(_B, dtype=jnp.int32) * _NPB)[:, None, None])
    gidx = gidx.reshape(_M, _NSAMPLE)

    u = _dense_u(p, x, W)

    c = new_xyz.reshape(_M, 3) @ W[:3]

    gidx_pad = jnp.pad(gidx, ((0, _MP - _M), (0, 0)))
    gmx, gmn, gs1, gs2 = _sc_gather_reduce(u, gidx_pad)
    gmx, gmn, gs1, gs2 = gmx[:_M], gmn[:_M], gs1[:_M], gs2[:_M]

    cnt = _M * _NSAMPLE
    s1 = jnp.sum(gs1 - _NSAMPLE * c, axis=0)
    s2 = jnp.sum(gs2 - 2.0 * c * gs1 + _NSAMPLE * c * c, axis=0)
    mean = s1 / cnt
    var = s2 / cnt - mean * mean
    gamma = cond_gamma[condition]
    beta = cond_beta[condition]
    scale = gamma / jnp.sqrt(var + 1e-5)
    bias = beta - mean * scale
    gsel = jnp.where(scale >= 0, gmx, gmn)
    out = jax.nn.relu((gsel - c) * scale[None, :] + bias[None, :])

    n_p = new_xyz.reshape(_M, 3)
    n_o = jnp.array([(i + 1) * _MPB for i in range(_B)], jnp.int32)
    return (n_p, out, n_o)
